# Initial kernel scaffold; baseline (speedup 1.0000x reference)
import jax
import jax.numpy as jnp
from jax.experimental import pallas as pl


def kernel(query, document, q_lens, d_lens):
    raise NotImplementedError("write your pallas kernel here")


# speedup vs baseline: 3.2277x; 3.2277x over previous
import jax
import jax.numpy as jnp
from jax import lax
from jax.experimental import pallas as pl
from jax.experimental.pallas import tpu as pltpu
from jax.experimental.pallas import tpu_sc as plsc
import functools
import numpy as np

CLS_ID = 101
SEP_ID = 102
MAX_LENGTH = 4096
B = 16
LQ = 64
HALF = MAX_LENGTH // 2
NCHUNK = 2 * B


def _body(q_hbm, d_hbm, qlens_hbm, dlens_hbm,
          tok_hbm, mask_hbm, pid_hbm,
          q_v, d_v, qlens_v, dlens_v, tok_v, mask_v, pid_v):
    nc = 2
    wid = lax.axis_index("s") * nc + lax.axis_index("c")
    row = wid // 2
    half = wid % 2
    base = half * HALF

    pltpu.sync_copy(qlens_hbm, qlens_v)
    pltpu.sync_copy(dlens_hbm, dlens_v)
    pltpu.sync_copy(q_hbm.at[row], q_v)
    pltpu.sync_copy(d_hbm.at[row], d_v)

    row_v = jnp.full((16,), row, jnp.int32)
    qlen = plsc.load_gather(qlens_v, [row_v])
    dlen = plsc.load_gather(dlens_v, [row_v])
    dlen_eff = jnp.minimum(dlen, np.int32(MAX_LENGTH - 2) - qlen)
    qoff = qlen + np.int32(2)
    total = qoff + dlen_eff
    lane = lax.iota(jnp.int32, 16)

    def step(i, _):
        p = lane + base.astype(jnp.int32) + i * np.int32(16)
        q_idx = jnp.clip(p - np.int32(1), np.int32(0), np.int32(LQ - 1))
        q_tok = plsc.load_gather(q_v, [q_idx])
        d_idx = jnp.clip(p - qoff, np.int32(0), np.int32(MAX_LENGTH - 1))
        d_tok = plsc.load_gather(d_v, [d_idx])
        in_seq = p < total
        tok = jnp.where(p == np.int32(0), np.int32(CLS_ID),
              jnp.where(p <= qlen, q_tok,
              jnp.where(p == qoff - np.int32(1), np.int32(SEP_ID),
              jnp.where(in_seq, d_tok, np.int32(0)))))
        mask = jnp.where(in_seq, np.float32(1.0), np.float32(0.0))
        pid = jnp.where(p <= qlen, p,
              jnp.where(in_seq, p - qlen - np.int32(1), np.int32(0)))
        off = i * np.int32(16)
        tok_v[pl.ds(off, 16)] = tok
        mask_v[pl.ds(off, 16)] = mask
        pid_v[pl.ds(off, 16)] = pid
        return 0

    lax.fori_loop(jnp.int32(0), jnp.int32(HALF // 16), step, 0)

    pltpu.sync_copy(tok_v, tok_hbm.at[wid])
    pltpu.sync_copy(mask_v, mask_hbm.at[wid])
    pltpu.sync_copy(pid_v, pid_hbm.at[wid])


@jax.jit
def _run(q32, d32, q_lens, d_lens):
    mesh = plsc.VectorSubcoreMesh(core_axis_name="c", subcore_axis_name="s")
    f = pl.kernel(
        _body,
        out_type=(
            jax.ShapeDtypeStruct((NCHUNK, HALF), jnp.int32),
            jax.ShapeDtypeStruct((NCHUNK, HALF), jnp.float32),
            jax.ShapeDtypeStruct((NCHUNK, HALF), jnp.int32),
        ),
        mesh=mesh,
        compiler_params=pltpu.CompilerParams(needs_layout_passes=False),
        scratch_types=[
            pltpu.VMEM((LQ,), jnp.int32),
            pltpu.VMEM((MAX_LENGTH,), jnp.int32),
            pltpu.VMEM((B,), jnp.int32),
            pltpu.VMEM((B,), jnp.int32),
            pltpu.VMEM((HALF,), jnp.int32),
            pltpu.VMEM((HALF,), jnp.float32),
            pltpu.VMEM((HALF,), jnp.int32),
        ],
    )
    return f(q32, d32, q_lens, d_lens)


def kernel(query, document, q_lens, d_lens):
    q32 = query.astype(jnp.int32)
    d32 = document.astype(jnp.int32)
    tok, mask, pid = _run(q32, d32, q_lens, d_lens)
    tok = tok.reshape(B, MAX_LENGTH).astype(query.dtype)
    mask = mask.reshape(B, MAX_LENGTH)
    pid = pid.reshape(B, MAX_LENGTH).astype(jnp.int64)
    return tok, mask, pid
